# baseline (speedup 1.0000x reference)
import jax
import jax.numpy as jnp
from jax import lax
from jax.experimental import pallas as pl
from jax.experimental.pallas import tpu as pltpu
from jax.experimental.pallas import tpu_sc as plsc

VOCAB = 1000001
EMBED_DIM = 32
BATCH = 16384

_NC = 2
_NS = 16
_NW = _NC * _NS
_CL = 1024
_NFULL = 30
_STRIPE = _NFULL * _CL + 512
_TAIL_LO = _NW * _STRIPE
_TAIL1 = 512
_TAIL2 = VOCAB - _TAIL_LO - _TAIL1
_OPAD = BATCH + 128
_B_MASK = (1 << 15) - 1
_SENTINEL = 40000 << 15
_TRASH = BATCH + 16


def _scan_gather_body(table_hbm, tail_hbm, idx_hbm, out_hbm, idxbuf, mlist,
                      vscr, buf0, buf1, rows_ring, tag_ring,
                      semc0, semc1, sem0, sem1, sem2, sem3):
    wid = lax.axis_index("s") * _NC + lax.axis_index("c")
    lo = wid * _STRIPE
    width = jnp.where(wid == _NW - 1, _STRIPE + _TAIL1 + _TAIL2, _STRIPE)
    iota16 = lax.iota(jnp.int32, 16)
    scat_sems = (sem0, sem1, sem2, sem3)

    pltpu.sync_copy(idx_hbm, idxbuf.at[pl.ds(0, BATCH)])

    def build(t, nm):
        v = idxbuf[pl.ds(t * 16, 16)]
        rel = v - lo
        key = lax.shift_right_logical(rel | (width - 1 - rel), 31)
        packed = (rel << 15) | (iota16 + t * 16)
        sv = plsc.sort_key_val(key, packed)[1]
        mlist[pl.ds(nm, 16)] = sv
        return nm + plsc.all_reduce_population_count(key == 0)[0]

    nm = lax.fori_loop(0, BATCH // 16, build, 0)
    mlist[pl.ds(nm, 16)] = jnp.full((16,), _SENTINEL, jnp.int32)
    nmv = (nm + 15) >> 4

    def count(t, cnts):
        g = mlist[pl.ds(t * 16, 16)] >> 27
        return tuple(
            cnts[gi] + plsc.all_reduce_population_count(g == gi)[0]
            for gi in range(8)
        )

    cnts = lax.fori_loop(0, nmv, count, (0,) * 8)
    gbase = [jnp.int32(0)]
    for gi in range(8):
        gbase.append(gbase[gi] + ((cnts[gi] + 15) & ~15) + 16)
    for gi in range(8):
        idxbuf[pl.ds(gbase[gi] + cnts[gi], 16)] = jnp.full(
            (16,), _SENTINEL, jnp.int32)
        idxbuf[pl.ds(gbase[gi] + cnts[gi] + 15, 16)] = jnp.full(
            (16,), _SENTINEL, jnp.int32)

    def split(t, ptrs):
        x = mlist[pl.ds(t * 16, 16)]
        g = x >> 27
        sv = plsc.sort_key_val(g, x)[1]
        new_ptrs = []
        start = 0
        for gi in range(8):
            cnt = plsc.all_reduce_population_count(g == gi)[0]
            rot = sv[(iota16 + start) & 15]
            idxbuf[pl.ds(ptrs[gi], 16)] = rot
            new_ptrs.append(ptrs[gi] + cnt)
            start = start + cnt
        return tuple(new_ptrs)

    lax.fori_loop(0, nmv, split, tuple(gbase[gi] for gi in range(8)))
    glen = tuple((cnts[gi] + 15) >> 4 for gi in range(8))

    def chunk_base(c):
        return pl.multiple_of(lo + c * _CL, 128)

    def wait_chunk(buf, semc):
        pltpu.make_async_copy(
            table_hbm.at[:, pl.ds(0, _CL)], buf, semc).wait()

    def wait_scat(si):
        pltpu.make_async_copy(
            rows_ring.at[si], out_hbm.at[tag_ring.at[si]], scat_sems[si]
        ).wait()

    def process_chunk(buf, c, cwidth, tg_in):
        coff = c * _CL
        gsel = jnp.minimum(c >> 2, 7)
        gb, gl = jnp.int32(0), jnp.int32(0)
        for k in range(8):
            gb = jnp.where(gsel == k, gbase[k], gb)
            gl = jnp.where(gsel == k, glen[k], gl)

        def filt(t, jc):
            x = idxbuf[pl.ds(gb + t * 16, 16)]
            d = (x >> 15) - coff
            key = lax.shift_right_logical(d | (cwidth - 1 - d), 31)
            sv = plsc.sort_key_val(key, x)[1]
            mlist[pl.ds(jc, 16)] = sv
            return jc + plsc.all_reduce_population_count(key == 0)[0]

        jc = lax.fori_loop(0, gl, filt, 0)
        dummy = ((coff << 15) | (BATCH + ((wid + c) & 127))).astype(jnp.int32)
        mlist[pl.ds(jc, 16)] = jnp.full((16,), 0, jnp.int32) + dummy
        nvec = (jc + 15) >> 4

        def ext(t, tg):
            x = mlist[pl.ds(t * 16, 16)]
            l = (x >> 15) - coff
            b = x & _B_MASK
            for si in range(4):

                @pl.when((tg & 3) == si)
                def _():
                    @pl.when(tg >= 4)
                    def _():
                        wait_scat(si)

                    for k in range(16):
                        lk = l[k]
                        col = jnp.full((16,), 0, jnp.int32) + lk
                        ga = plsc.load_gather(buf, [iota16, col])
                        gb = plsc.load_gather(buf, [iota16 + 16, col])
                        rows_ring[si, k, pl.ds(0, 16)] = ga
                        rows_ring[si, k, pl.ds(16, 16)] = gb
                    tag_ring[si, pl.ds(0, 16)] = b
                    pltpu.async_copy(
                        rows_ring.at[si], out_hbm.at[tag_ring.at[si]],
                        scat_sems[si])

            return tg + 1

        return lax.fori_loop(0, nvec, ext, tg_in)

    pltpu.async_copy(table_hbm.at[:, pl.ds(chunk_base(0), _CL)], buf0, semc0)
    pltpu.async_copy(table_hbm.at[:, pl.ds(chunk_base(1), _CL)], buf1, semc1)

    w30 = jnp.where(wid == _NW - 1, _CL, 512)

    def pair(cc, tg):
        c0 = cc * 2
        wait_chunk(buf0, semc0)
        tg = process_chunk(buf0, c0, _CL, tg)

        @pl.when(c0 + 2 <= _NFULL)
        def _():
            pltpu.async_copy(
                table_hbm.at[:, pl.ds(chunk_base(c0 + 2), _CL)], buf0, semc0)

        wait_chunk(buf1, semc1)
        tg = process_chunk(buf1, c0 + 1, _CL, tg)

        @pl.when(c0 + 3 <= _NFULL)
        def _():
            pltpu.async_copy(
                table_hbm.at[:, pl.ds(chunk_base(c0 + 3), _CL)], buf1, semc1)

        return tg

    tg = lax.fori_loop(0, _NFULL // 2, pair, 0)

    wait_chunk(buf0, semc0)
    tg = process_chunk(buf0, _NFULL, w30, tg)

    @pl.when(wid == _NW - 1)
    def _():
        pltpu.sync_copy(tail_hbm, buf1.at[:, pl.ds(0, 128)])

    tg = process_chunk(buf1, _NFULL + 1, _TAIL2, tg)

    for si in range(4):

        @pl.when(tg > si)
        def _():
            wait_scat(si)


@jax.jit
def _sc_scan_gather(table_t, tail, idx):
    mesh = plsc.VectorSubcoreMesh(core_axis_name="c", subcore_axis_name="s")
    return pl.kernel(
        _scan_gather_body,
        out_type=jax.ShapeDtypeStruct((_OPAD, 128), jnp.float32),
        mesh=mesh,
        scratch_types=[
            pltpu.VMEM((BATCH + 288,), jnp.int32),
            pltpu.VMEM((BATCH + 32,), jnp.int32),
            pltpu.VMEM((16,), jnp.int32),
            pltpu.VMEM((EMBED_DIM, _CL), jnp.float32),
            pltpu.VMEM((EMBED_DIM, _CL), jnp.float32),
            pltpu.VMEM((4, 16, 128), jnp.float32),
            pltpu.VMEM((4, 16), jnp.int32),
            pltpu.SemaphoreType.DMA,
            pltpu.SemaphoreType.DMA,
            pltpu.SemaphoreType.DMA,
            pltpu.SemaphoreType.DMA,
            pltpu.SemaphoreType.DMA,
            pltpu.SemaphoreType.DMA,
        ],
        compiler_params=pltpu.CompilerParams(
            use_tc_tiling_on_sc=True, needs_layout_passes=False),
    )(table_t, tail, idx)


def kernel(user_id, embedding_table):
    idx = jnp.asarray(user_id, jnp.int32)
    tail = jnp.pad(
        embedding_table[_TAIL_LO + _TAIL1 :],
        ((0, 128 - _TAIL2), (0, 0)),
    ).T
    out_padded = _sc_scan_gather(embedding_table.T, tail, idx)
    return lax.slice(out_padded, (0, 0), (BATCH, EMBED_DIM))


# speedup vs baseline: 1.0846x; 1.0846x over previous
import jax
import jax.numpy as jnp
from jax import lax
from jax.experimental import pallas as pl
from jax.experimental.pallas import tpu as pltpu
from jax.experimental.pallas import tpu_sc as plsc

VOCAB = 1000001
EMBED_DIM = 32
BATCH = 16384

_NC = 2
_NS = 16
_NW = _NC * _NS
_CL = 1024
_NFULL = 30
_STRIPE = _NFULL * _CL + 512
_TAIL_LO = _NW * _STRIPE
_TAIL1 = 512
_TAIL2 = VOCAB - _TAIL_LO - _TAIL1
_OPAD = BATCH + 128
_B_MASK = (1 << 15) - 1
_SENTINEL = 40000 << 15
_TRASH = BATCH + 16


def _scan_gather_body(table_hbm, tail_hbm, idx_hbm, out_hbm, idxbuf, mlist,
                      vscr, buf0, buf1, rows_ring, tag_ring,
                      semc0, semc1, sem0, sem1, sem2, sem3):
    wid = lax.axis_index("s") * _NC + lax.axis_index("c")
    lo = wid * _STRIPE
    width = jnp.where(wid == _NW - 1, _STRIPE + _TAIL1 + _TAIL2, _STRIPE)
    iota16 = lax.iota(jnp.int32, 16)
    scat_sems = (sem0, sem1, sem2, sem3)

    pltpu.sync_copy(idx_hbm, idxbuf.at[pl.ds(0, BATCH)])

    def build(q, nm):
        svs, pcs = [], []
        for u in range(4):
            t = q * 4 + u
            v = idxbuf[pl.ds(t * 16, 16)]
            rel = v - lo
            key = lax.shift_right_logical(rel | (width - 1 - rel), 31)
            packed = (rel << 15) | (iota16 + t * 16)
            svs.append(plsc.sort_key_val(key, packed)[1])
            pcs.append(plsc.all_reduce_population_count(key == 0)[0])
        for u in range(4):
            mlist[pl.ds(nm, 16)] = svs[u]
            nm = nm + pcs[u]
        return nm

    nm = lax.fori_loop(0, BATCH // 64, build, 0)
    mlist[pl.ds(nm, 16)] = jnp.full((16,), _SENTINEL, jnp.int32)
    nmv = (nm + 15) >> 4

    def count(t, cnts):
        g = mlist[pl.ds(t * 16, 16)] >> 27
        return tuple(
            cnts[gi] + plsc.all_reduce_population_count(g == gi)[0]
            for gi in range(8)
        )

    cnts = lax.fori_loop(0, nmv, count, (0,) * 8)
    gbase = [jnp.int32(0)]
    for gi in range(8):
        gbase.append(gbase[gi] + ((cnts[gi] + 15) & ~15) + 16)
    for gi in range(8):
        idxbuf[pl.ds(gbase[gi] + cnts[gi], 16)] = jnp.full(
            (16,), _SENTINEL, jnp.int32)
        idxbuf[pl.ds(gbase[gi] + cnts[gi] + 15, 16)] = jnp.full(
            (16,), _SENTINEL, jnp.int32)

    def split(t, ptrs):
        x = mlist[pl.ds(t * 16, 16)]
        g = x >> 27
        sv = plsc.sort_key_val(g, x)[1]
        new_ptrs = []
        start = 0
        for gi in range(8):
            cnt = plsc.all_reduce_population_count(g == gi)[0]
            rot = sv[(iota16 + start) & 15]
            idxbuf[pl.ds(ptrs[gi], 16)] = rot
            new_ptrs.append(ptrs[gi] + cnt)
            start = start + cnt
        return tuple(new_ptrs)

    lax.fori_loop(0, nmv, split, tuple(gbase[gi] for gi in range(8)))
    glen = tuple((cnts[gi] + 15) >> 4 for gi in range(8))

    def chunk_base(c):
        return pl.multiple_of(lo + c * _CL, 128)

    def wait_chunk(buf, semc):
        pltpu.make_async_copy(
            table_hbm.at[:, pl.ds(0, _CL)], buf, semc).wait()

    def wait_scat(si):
        pltpu.make_async_copy(
            rows_ring.at[si], out_hbm.at[tag_ring.at[si]], scat_sems[si]
        ).wait()

    def process_chunk(buf, c, cwidth, tg_in):
        coff = c * _CL
        gsel = jnp.minimum(c >> 2, 7)
        gb, gl = jnp.int32(0), jnp.int32(0)
        for k in range(8):
            gb = jnp.where(gsel == k, gbase[k], gb)
            gl = jnp.where(gsel == k, glen[k], gl)

        def filt(t, jc):
            x = idxbuf[pl.ds(gb + t * 16, 16)]
            d = (x >> 15) - coff
            key = lax.shift_right_logical(d | (cwidth - 1 - d), 31)
            sv = plsc.sort_key_val(key, x)[1]
            mlist[pl.ds(jc, 16)] = sv
            return jc + plsc.all_reduce_population_count(key == 0)[0]

        jc = lax.fori_loop(0, gl, filt, 0)
        dummy = ((coff << 15) | (BATCH + ((wid + c) & 127))).astype(jnp.int32)
        mlist[pl.ds(jc, 16)] = jnp.full((16,), 0, jnp.int32) + dummy
        nvec = (jc + 15) >> 4

        def ext(t, tg):
            x = mlist[pl.ds(t * 16, 16)]
            l = (x >> 15) - coff
            b = x & _B_MASK
            for si in range(4):

                @pl.when((tg & 3) == si)
                def _():
                    @pl.when(tg >= 4)
                    def _():
                        wait_scat(si)

                    for k in range(16):
                        lk = l[k]
                        col = jnp.full((16,), 0, jnp.int32) + lk
                        ga = plsc.load_gather(buf, [iota16, col])
                        gb = plsc.load_gather(buf, [iota16 + 16, col])
                        rows_ring[si, k, pl.ds(0, 16)] = ga
                        rows_ring[si, k, pl.ds(16, 16)] = gb
                    tag_ring[si, pl.ds(0, 16)] = b
                    pltpu.async_copy(
                        rows_ring.at[si], out_hbm.at[tag_ring.at[si]],
                        scat_sems[si])

            return tg + 1

        return lax.fori_loop(0, nvec, ext, tg_in)

    pltpu.async_copy(table_hbm.at[:, pl.ds(chunk_base(0), _CL)], buf0, semc0)
    pltpu.async_copy(table_hbm.at[:, pl.ds(chunk_base(1), _CL)], buf1, semc1)

    w30 = jnp.where(wid == _NW - 1, _CL, 512)

    def pair(cc, tg):
        c0 = cc * 2
        wait_chunk(buf0, semc0)
        tg = process_chunk(buf0, c0, _CL, tg)

        @pl.when(c0 + 2 <= _NFULL)
        def _():
            pltpu.async_copy(
                table_hbm.at[:, pl.ds(chunk_base(c0 + 2), _CL)], buf0, semc0)

        wait_chunk(buf1, semc1)
        tg = process_chunk(buf1, c0 + 1, _CL, tg)

        @pl.when(c0 + 3 <= _NFULL)
        def _():
            pltpu.async_copy(
                table_hbm.at[:, pl.ds(chunk_base(c0 + 3), _CL)], buf1, semc1)

        return tg

    tg = lax.fori_loop(0, _NFULL // 2, pair, 0)

    wait_chunk(buf0, semc0)
    tg = process_chunk(buf0, _NFULL, w30, tg)

    @pl.when(wid == _NW - 1)
    def _():
        pltpu.sync_copy(tail_hbm, buf1.at[:, pl.ds(0, 128)])

    tg = process_chunk(buf1, _NFULL + 1, _TAIL2, tg)

    for si in range(4):

        @pl.when(tg > si)
        def _():
            wait_scat(si)


@jax.jit
def _sc_scan_gather(table_t, tail, idx):
    mesh = plsc.VectorSubcoreMesh(core_axis_name="c", subcore_axis_name="s")
    return pl.kernel(
        _scan_gather_body,
        out_type=jax.ShapeDtypeStruct((_OPAD, 128), jnp.float32),
        mesh=mesh,
        scratch_types=[
            pltpu.VMEM((BATCH + 288,), jnp.int32),
            pltpu.VMEM((BATCH + 32,), jnp.int32),
            pltpu.VMEM((16,), jnp.int32),
            pltpu.VMEM((EMBED_DIM, _CL), jnp.float32),
            pltpu.VMEM((EMBED_DIM, _CL), jnp.float32),
            pltpu.VMEM((4, 16, 128), jnp.float32),
            pltpu.VMEM((4, 16), jnp.int32),
            pltpu.SemaphoreType.DMA,
            pltpu.SemaphoreType.DMA,
            pltpu.SemaphoreType.DMA,
            pltpu.SemaphoreType.DMA,
            pltpu.SemaphoreType.DMA,
            pltpu.SemaphoreType.DMA,
        ],
        compiler_params=pltpu.CompilerParams(
            use_tc_tiling_on_sc=True, needs_layout_passes=False),
    )(table_t, tail, idx)


def kernel(user_id, embedding_table):
    idx = jnp.asarray(user_id, jnp.int32)
    tail = jnp.pad(
        embedding_table[_TAIL_LO + _TAIL1 :],
        ((0, 128 - _TAIL2), (0, 0)),
    ).T
    out_padded = _sc_scan_gather(embedding_table.T, tail, idx)
    return lax.slice(out_padded, (0, 0), (BATCH, EMBED_DIM))


# speedup vs baseline: 1.0935x; 1.0082x over previous
import jax
import jax.numpy as jnp
from jax import lax
from jax.experimental import pallas as pl
from jax.experimental.pallas import tpu as pltpu
from jax.experimental.pallas import tpu_sc as plsc

VOCAB = 1000001
EMBED_DIM = 32
BATCH = 16384

_NC = 2
_NS = 16
_NW = _NC * _NS
_CL = 1024
_NFULL = 30
_STRIPE = _NFULL * _CL + 512
_TAIL_LO = _NW * _STRIPE
_TAIL1 = 512
_TAIL2 = VOCAB - _TAIL_LO - _TAIL1
_OPAD = BATCH + 128
_B_MASK = (1 << 15) - 1
_SENTINEL = 40000 << 15
_TRASH = BATCH + 16


def _scan_gather_body(table_hbm, tail_hbm, idx_hbm, out_hbm, idxbuf, mlist,
                      buf0, buf1, rows_ring, tag_ring,
                      semc0, semc1, sem0, sem1, sem2, sem3):
    wid = lax.axis_index("s") * _NC + lax.axis_index("c")
    lo = wid * _STRIPE
    width = jnp.where(wid == _NW - 1, _STRIPE + _TAIL1 + _TAIL2, _STRIPE)
    iota16 = lax.iota(jnp.int32, 16)
    scat_sems = (sem0, sem1, sem2, sem3)

    pltpu.sync_copy(idx_hbm, idxbuf.at[pl.ds(0, BATCH)])

    def build(q, nm):
        svs, pcs = [], []
        for u in range(4):
            t = q * 4 + u
            v = idxbuf[pl.ds(t * 16, 16)]
            rel = v - lo
            key = lax.shift_right_logical(rel | (width - 1 - rel), 31)
            packed = (rel << 15) | (iota16 + t * 16)
            svs.append(plsc.sort_key_val(key, packed)[1])
            pcs.append(plsc.all_reduce_population_count(key == 0)[0])
        for u in range(4):
            mlist[pl.ds(nm, 16)] = svs[u]
            nm = nm + pcs[u]
        return nm

    nm = lax.fori_loop(0, BATCH // 64, build, 0)
    mlist[pl.ds(nm, 16)] = jnp.full((16,), _SENTINEL, jnp.int32)
    nmv = (nm + 15) >> 4

    def count(t, cnts):
        g = mlist[pl.ds(t * 16, 16)] >> 27
        return tuple(
            cnts[gi] + plsc.all_reduce_population_count(g == gi)[0]
            for gi in range(8)
        )

    cnts = lax.fori_loop(0, nmv, count, (0,) * 8)
    gbase = [jnp.int32(0)]
    for gi in range(8):
        gbase.append(gbase[gi] + ((cnts[gi] + 15) & ~15) + 16)
    for gi in range(8):
        idxbuf[pl.ds(gbase[gi] + cnts[gi], 16)] = jnp.full(
            (16,), _SENTINEL, jnp.int32)
        idxbuf[pl.ds(gbase[gi] + cnts[gi] + 15, 16)] = jnp.full(
            (16,), _SENTINEL, jnp.int32)

    def split(t, ptrs):
        x = mlist[pl.ds(t * 16, 16)]
        g = x >> 27
        sv = plsc.sort_key_val(g, x)[1]
        new_ptrs = []
        start = 0
        for gi in range(8):
            cnt = plsc.all_reduce_population_count(g == gi)[0]
            rot = sv[(iota16 + start) & 15]
            idxbuf[pl.ds(ptrs[gi], 16)] = rot
            new_ptrs.append(ptrs[gi] + cnt)
            start = start + cnt
        return tuple(new_ptrs)

    lax.fori_loop(0, nmv, split, tuple(gbase[gi] for gi in range(8)))
    glen = tuple((cnts[gi] + 15) >> 4 for gi in range(8))

    def chunk_base(c):
        return pl.multiple_of(lo + c * _CL, 128)

    def wait_chunk(buf, semc):
        pltpu.make_async_copy(
            table_hbm.at[:, pl.ds(0, _CL)], buf, semc).wait()

    def wait_scat(si):
        pltpu.make_async_copy(
            rows_ring.at[si], out_hbm.at[tag_ring.at[si]], scat_sems[si]
        ).wait()

    def process_chunk(buf, c, cwidth, tg_in):
        coff = c * _CL
        gsel = jnp.minimum(c >> 2, 7)
        gb, gl = jnp.int32(0), jnp.int32(0)
        for k in range(8):
            gb = jnp.where(gsel == k, gbase[k], gb)
            gl = jnp.where(gsel == k, glen[k], gl)

        def filt(q, jc):
            svs, pcs = [], []
            for u in range(2):
                x = idxbuf[pl.ds(gb + (q * 2 + u) * 16, 16)]
                d = (x >> 15) - coff
                key = lax.shift_right_logical(d | (cwidth - 1 - d), 31)
                svs.append(plsc.sort_key_val(key, x)[1])
                pcs.append(plsc.all_reduce_population_count(key == 0)[0])
            for u in range(2):
                mlist[pl.ds(jc, 16)] = svs[u]
                jc = jc + pcs[u]
            return jc

        jc = lax.fori_loop(0, (gl + 1) >> 1, filt, 0)
        dummy = ((coff << 15) | (BATCH + ((wid + c) & 127))).astype(jnp.int32)
        mlist[pl.ds(jc, 16)] = jnp.full((16,), 0, jnp.int32) + dummy
        nvec = (jc + 15) >> 4

        def ext(t, tg):
            x = mlist[pl.ds(t * 16, 16)]
            l = (x >> 15) - coff
            b = x & _B_MASK
            for si in range(4):

                @pl.when((tg & 3) == si)
                def _():
                    @pl.when(tg >= 4)
                    def _():
                        wait_scat(si)

                    for k in range(16):
                        lk = l[k]
                        col = jnp.full((16,), 0, jnp.int32) + lk
                        ga = plsc.load_gather(buf, [iota16, col])
                        gb = plsc.load_gather(buf, [iota16 + 16, col])
                        rows_ring[si, k, pl.ds(0, 16)] = ga
                        rows_ring[si, k, pl.ds(16, 16)] = gb
                    tag_ring[si, pl.ds(0, 16)] = b
                    pltpu.async_copy(
                        rows_ring.at[si], out_hbm.at[tag_ring.at[si]],
                        scat_sems[si])

            return tg + 1

        return lax.fori_loop(0, nvec, ext, tg_in)

    pltpu.async_copy(table_hbm.at[:, pl.ds(chunk_base(0), _CL)], buf0, semc0)
    pltpu.async_copy(table_hbm.at[:, pl.ds(chunk_base(1), _CL)], buf1, semc1)

    w30 = jnp.where(wid == _NW - 1, _CL, 512)

    def pair(cc, tg):
        c0 = cc * 2
        wait_chunk(buf0, semc0)
        tg = process_chunk(buf0, c0, _CL, tg)

        @pl.when(c0 + 2 <= _NFULL)
        def _():
            pltpu.async_copy(
                table_hbm.at[:, pl.ds(chunk_base(c0 + 2), _CL)], buf0, semc0)

        wait_chunk(buf1, semc1)
        tg = process_chunk(buf1, c0 + 1, _CL, tg)

        @pl.when(c0 + 3 <= _NFULL)
        def _():
            pltpu.async_copy(
                table_hbm.at[:, pl.ds(chunk_base(c0 + 3), _CL)], buf1, semc1)

        return tg

    tg = lax.fori_loop(0, _NFULL // 2, pair, 0)

    wait_chunk(buf0, semc0)
    tg = process_chunk(buf0, _NFULL, w30, tg)

    @pl.when(wid == _NW - 1)
    def _():
        pltpu.sync_copy(tail_hbm, buf1.at[:, pl.ds(0, 128)])

    tg = process_chunk(buf1, _NFULL + 1, _TAIL2, tg)

    for si in range(4):

        @pl.when(tg > si)
        def _():
            wait_scat(si)


@jax.jit
def _sc_scan_gather(table_t, tail, idx):
    mesh = plsc.VectorSubcoreMesh(core_axis_name="c", subcore_axis_name="s")
    return pl.kernel(
        _scan_gather_body,
        out_type=jax.ShapeDtypeStruct((_OPAD, 128), jnp.float32),
        mesh=mesh,
        scratch_types=[
            pltpu.VMEM((BATCH + 288,), jnp.int32),
            pltpu.VMEM((BATCH + 32,), jnp.int32),
            pltpu.VMEM((EMBED_DIM, _CL), jnp.float32),
            pltpu.VMEM((EMBED_DIM, _CL), jnp.float32),
            pltpu.VMEM((4, 16, 128), jnp.float32),
            pltpu.VMEM((4, 16), jnp.int32),
            pltpu.SemaphoreType.DMA,
            pltpu.SemaphoreType.DMA,
            pltpu.SemaphoreType.DMA,
            pltpu.SemaphoreType.DMA,
            pltpu.SemaphoreType.DMA,
            pltpu.SemaphoreType.DMA,
        ],
        compiler_params=pltpu.CompilerParams(
            use_tc_tiling_on_sc=True, needs_layout_passes=False),
    )(table_t, tail, idx)


def kernel(user_id, embedding_table):
    idx = jnp.asarray(user_id, jnp.int32)
    tail = jnp.pad(
        embedding_table[_TAIL_LO + _TAIL1 :],
        ((0, 128 - _TAIL2), (0, 0)),
    ).T
    out_padded = _sc_scan_gather(embedding_table.T, tail, idx)
    return lax.slice(out_padded, (0, 0), (BATCH, EMBED_DIM))


# speedup vs baseline: 1.0971x; 1.0033x over previous
import jax
import jax.numpy as jnp
from jax import lax
from jax.experimental import pallas as pl
from jax.experimental.pallas import tpu as pltpu
from jax.experimental.pallas import tpu_sc as plsc

VOCAB = 1000001
EMBED_DIM = 32
BATCH = 16384

_NC = 2
_NS = 16
_NW = _NC * _NS
_CL = 1024
_NFULL = 30
_STRIPE = _NFULL * _CL + 512
_TAIL_LO = _NW * _STRIPE
_TAIL1 = 512
_TAIL2 = VOCAB - _TAIL_LO - _TAIL1
_OPAD = BATCH + 128
_B_MASK = (1 << 15) - 1
_SENTINEL = 40000 << 15
_TRASH = BATCH + 16


def _scan_gather_body(table_hbm, tail_hbm, idx_hbm, out_hbm, idxbuf, mlist,
                      buf0, buf1, rows_ring, tag_ring,
                      semc0, semc1, sem0, sem1, sem2, sem3):
    wid = lax.axis_index("s") * _NC + lax.axis_index("c")
    lo = wid * _STRIPE
    width = jnp.where(wid == _NW - 1, _STRIPE + _TAIL1 + _TAIL2, _STRIPE)
    iota16 = lax.iota(jnp.int32, 16)
    scat_sems = (sem0, sem1, sem2, sem3)

    def chunk_base(c):
        return pl.multiple_of(lo + c * _CL, 128)

    pltpu.async_copy(table_hbm.at[:, pl.ds(chunk_base(0), _CL)], buf0, semc0)
    pltpu.async_copy(table_hbm.at[:, pl.ds(chunk_base(1), _CL)], buf1, semc1)
    pltpu.sync_copy(idx_hbm, idxbuf.at[pl.ds(0, BATCH)])

    def build(q, nm):
        svs, pcs = [], []
        for u in range(4):
            t = q * 4 + u
            v = idxbuf[pl.ds(t * 16, 16)]
            rel = v - lo
            key = lax.shift_right_logical(rel | (width - 1 - rel), 31)
            packed = (rel << 15) | (iota16 + t * 16)
            svs.append(plsc.sort_key_val(key, packed)[1])
            pcs.append(plsc.all_reduce_population_count(key == 0)[0])
        for u in range(4):
            mlist[pl.ds(nm, 16)] = svs[u]
            nm = nm + pcs[u]
        return nm

    nm = lax.fori_loop(0, BATCH // 64, build, 0)
    mlist[pl.ds(nm, 16)] = jnp.full((16,), _SENTINEL, jnp.int32)
    nmv = (nm + 15) >> 4

    def count(t, cnts):
        g = mlist[pl.ds(t * 16, 16)] >> 27
        return tuple(
            cnts[gi] + plsc.all_reduce_population_count(g == gi)[0]
            for gi in range(8)
        )

    cnts = lax.fori_loop(0, nmv, count, (0,) * 8)
    gbase = [jnp.int32(0)]
    for gi in range(8):
        gbase.append(gbase[gi] + ((cnts[gi] + 15) & ~15) + 16)
    for gi in range(8):
        idxbuf[pl.ds(gbase[gi] + cnts[gi], 16)] = jnp.full(
            (16,), _SENTINEL, jnp.int32)
        idxbuf[pl.ds(gbase[gi] + cnts[gi] + 15, 16)] = jnp.full(
            (16,), _SENTINEL, jnp.int32)

    def split(t, ptrs):
        x = mlist[pl.ds(t * 16, 16)]
        g = x >> 27
        sv = plsc.sort_key_val(g, x)[1]
        new_ptrs = []
        start = 0
        for gi in range(8):
            cnt = plsc.all_reduce_population_count(g == gi)[0]
            rot = sv[(iota16 + start) & 15]
            idxbuf[pl.ds(ptrs[gi], 16)] = rot
            new_ptrs.append(ptrs[gi] + cnt)
            start = start + cnt
        return tuple(new_ptrs)

    lax.fori_loop(0, nmv, split, tuple(gbase[gi] for gi in range(8)))
    glen = tuple((cnts[gi] + 15) >> 4 for gi in range(8))

    def wait_chunk(buf, semc):
        pltpu.make_async_copy(
            table_hbm.at[:, pl.ds(0, _CL)], buf, semc).wait()

    def wait_scat(si):
        pltpu.make_async_copy(
            rows_ring.at[si], out_hbm.at[tag_ring.at[si]], scat_sems[si]
        ).wait()

    def process_chunk(buf, c, cwidth, tg_in):
        coff = c * _CL
        gsel = jnp.minimum(c >> 2, 7)
        gb, gl = jnp.int32(0), jnp.int32(0)
        for k in range(8):
            gb = jnp.where(gsel == k, gbase[k], gb)
            gl = jnp.where(gsel == k, glen[k], gl)

        def filt(q, jc):
            svs, pcs = [], []
            for u in range(2):
                x = idxbuf[pl.ds(gb + (q * 2 + u) * 16, 16)]
                d = (x >> 15) - coff
                key = lax.shift_right_logical(d | (cwidth - 1 - d), 31)
                svs.append(plsc.sort_key_val(key, x)[1])
                pcs.append(plsc.all_reduce_population_count(key == 0)[0])
            for u in range(2):
                mlist[pl.ds(jc, 16)] = svs[u]
                jc = jc + pcs[u]
            return jc

        jc = lax.fori_loop(0, (gl + 1) >> 1, filt, 0)
        dummy = ((coff << 15) | (BATCH + ((wid + c) & 127))).astype(jnp.int32)
        mlist[pl.ds(jc, 16)] = jnp.full((16,), 0, jnp.int32) + dummy
        nvec = (jc + 15) >> 4

        def ext(t, tg):
            x = mlist[pl.ds(t * 16, 16)]
            l = (x >> 15) - coff
            b = x & _B_MASK
            for si in range(4):

                @pl.when((tg & 3) == si)
                def _():
                    @pl.when(tg >= 4)
                    def _():
                        wait_scat(si)

                    for k in range(16):
                        lk = l[k]
                        col = jnp.full((16,), 0, jnp.int32) + lk
                        ga = plsc.load_gather(buf, [iota16, col])
                        gb = plsc.load_gather(buf, [iota16 + 16, col])
                        rows_ring[si, k, pl.ds(0, 16)] = ga
                        rows_ring[si, k, pl.ds(16, 16)] = gb
                    tag_ring[si, pl.ds(0, 16)] = b
                    pltpu.async_copy(
                        rows_ring.at[si], out_hbm.at[tag_ring.at[si]],
                        scat_sems[si])

            return tg + 1

        return lax.fori_loop(0, nvec, ext, tg_in)

    w30 = jnp.where(wid == _NW - 1, _CL, 512)

    def pair(cc, tg):
        c0 = cc * 2
        wait_chunk(buf0, semc0)
        tg = process_chunk(buf0, c0, _CL, tg)

        @pl.when(c0 + 2 <= _NFULL)
        def _():
            pltpu.async_copy(
                table_hbm.at[:, pl.ds(chunk_base(c0 + 2), _CL)], buf0, semc0)

        wait_chunk(buf1, semc1)
        tg = process_chunk(buf1, c0 + 1, _CL, tg)

        @pl.when(c0 + 3 <= _NFULL)
        def _():
            pltpu.async_copy(
                table_hbm.at[:, pl.ds(chunk_base(c0 + 3), _CL)], buf1, semc1)

        return tg

    tg = lax.fori_loop(0, _NFULL // 2, pair, 0)

    wait_chunk(buf0, semc0)
    tg = process_chunk(buf0, _NFULL, w30, tg)

    @pl.when(wid == _NW - 1)
    def _():
        pltpu.sync_copy(tail_hbm, buf1.at[:, pl.ds(0, 128)])

    tg = process_chunk(buf1, _NFULL + 1, _TAIL2, tg)

    for si in range(4):

        @pl.when(tg > si)
        def _():
            wait_scat(si)


@jax.jit
def _sc_scan_gather(table_t, tail, idx):
    mesh = plsc.VectorSubcoreMesh(core_axis_name="c", subcore_axis_name="s")
    return pl.kernel(
        _scan_gather_body,
        out_type=jax.ShapeDtypeStruct((_OPAD, 128), jnp.float32),
        mesh=mesh,
        scratch_types=[
            pltpu.VMEM((BATCH + 288,), jnp.int32),
            pltpu.VMEM((BATCH + 32,), jnp.int32),
            pltpu.VMEM((EMBED_DIM, _CL), jnp.float32),
            pltpu.VMEM((EMBED_DIM, _CL), jnp.float32),
            pltpu.VMEM((4, 16, 128), jnp.float32),
            pltpu.VMEM((4, 16), jnp.int32),
            pltpu.SemaphoreType.DMA,
            pltpu.SemaphoreType.DMA,
            pltpu.SemaphoreType.DMA,
            pltpu.SemaphoreType.DMA,
            pltpu.SemaphoreType.DMA,
            pltpu.SemaphoreType.DMA,
        ],
        compiler_params=pltpu.CompilerParams(
            use_tc_tiling_on_sc=True, needs_layout_passes=False),
    )(table_t, tail, idx)


def kernel(user_id, embedding_table):
    idx = jnp.asarray(user_id, jnp.int32)
    tail = jnp.pad(
        embedding_table[_TAIL_LO + _TAIL1 :],
        ((0, 128 - _TAIL2), (0, 0)),
    ).T
    out_padded = _sc_scan_gather(embedding_table.T, tail, idx)
    return lax.slice(out_padded, (0, 0), (BATCH, EMBED_DIM))
